# baseline (speedup 1.0000x reference)
import functools

import jax
import jax.numpy as jnp
from jax.experimental import pallas as pl
import jax.experimental.pallas.tpu as pltpu

ROWS = 128
VOCAB = 100000
BLK = 4096
NUM_BLOCKS = -(-VOCAB // BLK)


def _argmax_body(x_ref, out_ref, vmax_ref, vidx_ref):
    i = pl.program_id(0)
    x = x_ref[...]
    col = jax.lax.broadcasted_iota(jnp.int32, x.shape, 1)
    valid = (col + i * BLK) < VOCAB
    x = jnp.where(valid, x, -jnp.inf)
    bmax = jnp.max(x, axis=1, keepdims=True)
    bidx = jnp.argmax(x, axis=1).reshape(ROWS, 1).astype(jnp.int32) + i * BLK

    @pl.when(i == 0)
    def _():
        vmax_ref[...] = bmax
        vidx_ref[...] = bidx

    @pl.when(i > 0)
    def _():
        better = bmax > vmax_ref[...]
        vidx_ref[...] = jnp.where(better, bidx, vidx_ref[...])
        vmax_ref[...] = jnp.where(better, bmax, vmax_ref[...])

    @pl.when(i == NUM_BLOCKS - 1)
    def _():
        out_ref[...] = vidx_ref[...]


@jax.jit
def _argmax_pallas(m_logits):
    out = pl.pallas_call(
        _argmax_body,
        grid=(NUM_BLOCKS,),
        in_specs=[pl.BlockSpec((ROWS, BLK), lambda i: (0, i))],
        out_specs=pl.BlockSpec((ROWS, 1), lambda i: (0, 0)),
        out_shape=jax.ShapeDtypeStruct((ROWS, 1), jnp.int32),
        scratch_shapes=[
            pltpu.VMEM((ROWS, 1), jnp.float32),
            pltpu.VMEM((ROWS, 1), jnp.int32),
        ],
    )(m_logits)
    return out


def kernel(m_logits):
    token = _argmax_pallas(m_logits.astype(jnp.float32))
    return token.astype(jnp.int64)


# speedup vs baseline: 1.0954x; 1.0954x over previous
import jax
import jax.numpy as jnp
from jax.experimental import pallas as pl
import jax.experimental.pallas.tpu as pltpu

ROWS = 128
VOCAB = 100000
W = 2048
CHUNKS_PER_BLK = 4
BLK = W * CHUNKS_PER_BLK
NUM_BLOCKS = -(-VOCAB // BLK)


def _argmax_body(x_ref, out_ref, vmax_ref, vchunk_ref):
    i = pl.program_id(0)

    def fold(k, chunk):
        cid = i * CHUNKS_PER_BLK + k
        better = chunk > vmax_ref[...]
        vchunk_ref[...] = jnp.where(better, cid, vchunk_ref[...])
        vmax_ref[...] = jnp.where(better, chunk, vmax_ref[...])

    @pl.when(i == 0)
    def _():
        vmax_ref[...] = x_ref[:, :W]
        vchunk_ref[...] = jnp.zeros((ROWS, W), jnp.int32)
        for k in range(1, CHUNKS_PER_BLK):
            fold(k, x_ref[:, k * W:(k + 1) * W])

    @pl.when(jnp.logical_and(i > 0, i < NUM_BLOCKS - 1))
    def _():
        for k in range(CHUNKS_PER_BLK):
            fold(k, x_ref[:, k * W:(k + 1) * W])

    @pl.when(i == NUM_BLOCKS - 1)
    def _():
        base = i * BLK
        for k in range(CHUNKS_PER_BLK):
            chunk = x_ref[:, k * W:(k + 1) * W]
            col = jax.lax.broadcasted_iota(jnp.int32, (ROWS, W), 1)
            valid = (col + (base + k * W)) < VOCAB
            fold(k, jnp.where(valid, chunk, -jnp.inf))

        vmax = vmax_ref[...]
        m = jnp.max(vmax, axis=1, keepdims=True)
        slot = jax.lax.broadcasted_iota(jnp.int32, (ROWS, W), 1)
        gcol = vchunk_ref[...] * W + slot
        cand = jnp.where(vmax == m, gcol, jnp.int32(2**31 - 1))
        out_ref[...] = jnp.min(cand, axis=1, keepdims=True)


@jax.jit
def _argmax_pallas(m_logits):
    return pl.pallas_call(
        _argmax_body,
        grid=(NUM_BLOCKS,),
        in_specs=[pl.BlockSpec((ROWS, BLK), lambda i: (0, i))],
        out_specs=pl.BlockSpec((ROWS, 1), lambda i: (0, 0)),
        out_shape=jax.ShapeDtypeStruct((ROWS, 1), jnp.int32),
        scratch_shapes=[
            pltpu.VMEM((ROWS, W), jnp.float32),
            pltpu.VMEM((ROWS, W), jnp.int32),
        ],
    )(m_logits)


def kernel(m_logits):
    token = _argmax_pallas(m_logits.astype(jnp.float32))
    return token.astype(jnp.int64)


# speedup vs baseline: 1.1602x; 1.0591x over previous
import jax
import jax.numpy as jnp
from jax.experimental import pallas as pl
import jax.experimental.pallas.tpu as pltpu

ROWS = 128
VOCAB = 100000
W = 128
CHUNKS_PER_BLK = 64
BLK = W * CHUNKS_PER_BLK
NUM_BLOCKS = -(-VOCAB // BLK)
TAIL_COLS = VOCAB - (NUM_BLOCKS - 1) * BLK
TAIL_FULL_CHUNKS = TAIL_COLS // W
TAIL_REM = TAIL_COLS - TAIL_FULL_CHUNKS * W


def _fold_chunks(x_ref, vmax, vchunk, ks, i, masked_k=None):
    for k in ks:
        chunk = x_ref[:, k * W:(k + 1) * W]
        if masked_k is not None and k == masked_k:
            col = jax.lax.broadcasted_iota(jnp.int32, (ROWS, W), 1)
            chunk = jnp.where(col < TAIL_REM, chunk, -jnp.inf)
        cid = i * CHUNKS_PER_BLK + k
        better = chunk > vmax
        vchunk = jnp.where(better, cid, vchunk)
        vmax = jnp.where(better, chunk, vmax)
    return vmax, vchunk


def _argmax_body(x_ref, out_ref, vmax_ref, vchunk_ref):
    i = pl.program_id(0)

    @pl.when(i == 0)
    def _():
        vmax = x_ref[:, :W]
        vchunk = jnp.zeros((ROWS, W), jnp.int32)
        vmax, vchunk = _fold_chunks(x_ref, vmax, vchunk,
                                    range(1, CHUNKS_PER_BLK), 0)
        vmax_ref[...] = vmax
        vchunk_ref[...] = vchunk

    @pl.when(jnp.logical_and(i > 0, i < NUM_BLOCKS - 1))
    def _():
        vmax, vchunk = _fold_chunks(x_ref, vmax_ref[...], vchunk_ref[...],
                                    range(CHUNKS_PER_BLK), i)
        vmax_ref[...] = vmax
        vchunk_ref[...] = vchunk

    @pl.when(i == NUM_BLOCKS - 1)
    def _():
        vmax, vchunk = _fold_chunks(
            x_ref, vmax_ref[...], vchunk_ref[...],
            range(TAIL_FULL_CHUNKS + (1 if TAIL_REM else 0)),
            NUM_BLOCKS - 1,
            masked_k=TAIL_FULL_CHUNKS if TAIL_REM else None)

        m = jnp.max(vmax, axis=1, keepdims=True)
        slot = jax.lax.broadcasted_iota(jnp.int32, (ROWS, W), 1)
        gcol = vchunk * W + slot
        cand = jnp.where(vmax == m, gcol, jnp.int32(2**31 - 1))
        out_ref[...] = jnp.min(cand, axis=1, keepdims=True)


@jax.jit
def _argmax_pallas(m_logits):
    return pl.pallas_call(
        _argmax_body,
        grid=(NUM_BLOCKS,),
        in_specs=[pl.BlockSpec((ROWS, BLK), lambda i: (0, i))],
        out_specs=pl.BlockSpec((ROWS, 1), lambda i: (0, 0)),
        out_shape=jax.ShapeDtypeStruct((ROWS, 1), jnp.int32),
        scratch_shapes=[
            pltpu.VMEM((ROWS, W), jnp.float32),
            pltpu.VMEM((ROWS, W), jnp.int32),
        ],
    )(m_logits)


def kernel(m_logits):
    token = _argmax_pallas(m_logits.astype(jnp.float32))
    return token.astype(jnp.int64)
